# Initial kernel scaffold; baseline (speedup 1.0000x reference)
import jax
import jax.numpy as jnp
from jax.experimental import pallas as pl


def kernel(heatmap):
    raise NotImplementedError("write your pallas kernel here")


# speedup vs baseline: 10.9645x; 10.9645x over previous
import jax
import jax.numpy as jnp
from jax.experimental import pallas as pl
from jax.experimental.pallas import tpu as pltpu

_N_TARGETS = 8
_RADIUS = 4
_RECLEN = 2 * _RADIUS + 1
_H = 512
_W = 512
_PH = _H - _RECLEN + 1
_PW = _W - _RECLEN + 1


def _pool_sums(hm, nrows):
    s1 = hm[:, 0:_PW]
    for j in range(1, _RECLEN):
        s1 = s1 + hm[:, j:j + _PW]
    out = s1[0:nrows, :]
    for i in range(1, _RECLEN):
        out = out + s1[i:i + nrows, :]
    return out


def _peaks_body(hm_ref, out_ref, hm_s, p_s):
    hm_s[:, :] = hm_ref[0, 0]
    p_s[:, :] = jnp.full((_H, _W), -1.0, dtype=jnp.float32)
    p_s[0:_PH, 0:_PW] = _pool_sums(hm_s[:, :], _PH)

    row_i = jax.lax.broadcasted_iota(jnp.int32, (_H, _W), 0)
    col_i = jax.lax.broadcasted_iota(jnp.int32, (_H, _W), 1)
    lin = row_i * _W + col_i
    big = jnp.int32(1 << 30)

    for t in range(_N_TARGETS):
        p = p_s[:, :]
        m = jnp.max(p)
        idx = jnp.min(jnp.where(p == m, lin, big))
        r0 = idx // _W
        c0 = idx - r0 * _W
        out_ref[0, t, 0] = c0 + _RADIUS
        out_ref[0, t, 1] = r0 + _RADIUS

        ra = (r0 // 8) * 8
        band = hm_s[pl.ds(ra, 16), :]
        br = jax.lax.broadcasted_iota(jnp.int32, (16, _W), 0) + ra
        bc = jax.lax.broadcasted_iota(jnp.int32, (16, _W), 1)
        inside = (
            (br >= r0) & (br < r0 + _RECLEN)
            & (bc >= c0) & (bc < c0 + _RECLEN)
        )
        hm_s[pl.ds(ra, 16), :] = jnp.where(inside, 0.0, band)

        pr = jnp.minimum((jnp.maximum(r0 - 8, 0) // 8) * 8, _PH - 32)
        hband = hm_s[pl.ds(pr, 40), :]
        p_s[pl.ds(pr, 32), 0:_PW] = _pool_sums(hband, 32)


@jax.jit
def kernel(heatmap):
    B = heatmap.shape[0]
    out = pl.pallas_call(
        _peaks_body,
        grid=(B,),
        in_specs=[
            pl.BlockSpec((1, 1, _H, _W), lambda b: (b, 0, 0, 0)),
        ],
        out_specs=pl.BlockSpec(
            (1, _N_TARGETS, 2), lambda b: (b, 0, 0),
            memory_space=pltpu.SMEM,
        ),
        out_shape=jax.ShapeDtypeStruct((B, _N_TARGETS, 2), jnp.int32),
        scratch_shapes=[
            pltpu.VMEM((_H, _W), jnp.float32),
            pltpu.VMEM((_H, _W), jnp.float32),
        ],
        compiler_params=pltpu.CompilerParams(
            dimension_semantics=("arbitrary",),
        ),
    )(heatmap)
    return out


# speedup vs baseline: 10.9666x; 1.0002x over previous
import jax
import jax.numpy as jnp
from jax.experimental import pallas as pl
from jax.experimental.pallas import tpu as pltpu

_N_TARGETS = 8
_RADIUS = 4
_RECLEN = 2 * _RADIUS + 1
_H = 512
_W = 512
_PH = _H - _RECLEN + 1
_PW = _W - _RECLEN + 1


def _pool_sums(hm, nrows):
    s1 = hm[:, 0:_PW]
    for j in range(1, _RECLEN):
        s1 = s1 + hm[:, j:j + _PW]
    out = s1[0:nrows, :]
    for i in range(1, _RECLEN):
        out = out + s1[i:i + nrows, :]
    return out


def _peaks_body(hm_ref, out_ref, hm_s, p_s):
    hm_s[:, :] = hm_ref[0, 0]
    p_s[:, :] = jnp.full((_H, _W), -1.0, dtype=jnp.float32)
    p_s[0:_PH, 0:_PW] = _pool_sums(hm_s[:, :], _PH)

    row_i = jax.lax.broadcasted_iota(jnp.int32, (_H, _W), 0)
    col_i = jax.lax.broadcasted_iota(jnp.int32, (_H, _W), 1)
    lin = row_i * _W + col_i
    big = jnp.int32(1 << 30)

    for t in range(_N_TARGETS):
        p = p_s[:, :]
        m = jnp.max(p)
        idx = jnp.min(jnp.where(p == m, lin, big))
        r0 = idx // _W
        c0 = idx - r0 * _W
        out_ref[0, t, 0] = c0 + _RADIUS
        out_ref[0, t, 1] = r0 + _RADIUS

        ra = (r0 // 8) * 8
        band = hm_s[pl.ds(ra, 16), :]
        br = jax.lax.broadcasted_iota(jnp.int32, (16, _W), 0) + ra
        bc = jax.lax.broadcasted_iota(jnp.int32, (16, _W), 1)
        inside = (
            (br >= r0) & (br < r0 + _RECLEN)
            & (bc >= c0) & (bc < c0 + _RECLEN)
        )
        hm_s[pl.ds(ra, 16), :] = jnp.where(inside, 0.0, band)

        pr = jnp.minimum((jnp.maximum(r0 - 8, 0) // 8) * 8, _PH - 32)
        hband = hm_s[pl.ds(pr, 40), :]
        p_s[pl.ds(pr, 32), 0:_PW] = _pool_sums(hband, 32)


@jax.jit
def kernel(heatmap):
    B = heatmap.shape[0]
    out = pl.pallas_call(
        _peaks_body,
        grid=(B,),
        in_specs=[
            pl.BlockSpec((1, 1, _H, _W), lambda b: (b, 0, 0, 0)),
        ],
        out_specs=pl.BlockSpec(
            (1, _N_TARGETS, 2), lambda b: (b, 0, 0),
            memory_space=pltpu.SMEM,
        ),
        out_shape=jax.ShapeDtypeStruct((B, _N_TARGETS, 2), jnp.int32),
        scratch_shapes=[
            pltpu.VMEM((_H, _W), jnp.float32),
            pltpu.VMEM((_H, _W), jnp.float32),
        ],
        compiler_params=pltpu.CompilerParams(
            dimension_semantics=("parallel",),
        ),
    )(heatmap)
    return out


# speedup vs baseline: 18.7109x; 1.7062x over previous
import jax
import jax.numpy as jnp
from jax.experimental import pallas as pl
from jax.experimental.pallas import tpu as pltpu

_N_TARGETS = 8
_RADIUS = 4
_RECLEN = 2 * _RADIUS + 1
_H = 512
_W = 512
_PH = _H - _RECLEN + 1
_PW = _W - _RECLEN + 1
_PAIR = 4


def _pool_sums(hm, nrows):
    nr = hm.shape[0]
    r2 = hm + pltpu.roll(hm, _W - 1, 1)
    r4 = r2 + pltpu.roll(r2, _W - 2, 1)
    r8 = r4 + pltpu.roll(r4, _W - 4, 1)
    r9 = r8 + pltpu.roll(hm, _W - 8, 1)
    t2 = r9 + pltpu.roll(r9, nr - 1, 0)
    t4 = t2 + pltpu.roll(t2, nr - 2, 0)
    t8 = t4 + pltpu.roll(t4, nr - 4, 0)
    t9 = t8 + pltpu.roll(r9, nr - 8, 0)
    return t9[0:nrows, 0:_PW]


def _peaks_body(*refs):
    hm_refs = refs[:_PAIR]
    out_ref = refs[_PAIR]
    p_ss = refs[_PAIR + 1:2 * _PAIR + 1]
    rm_ss = refs[2 * _PAIR + 1:]

    big = jnp.int32(1 << 30)
    rowi = jax.lax.broadcasted_iota(jnp.int32, (_H, 1), 0)
    srow = jax.lax.broadcasted_iota(jnp.int32, (8, _W), 0)
    scol = jax.lax.broadcasted_iota(jnp.int32, (8, _W), 1)

    for i in range(_PAIR):
        p_s, rm_s = p_ss[i], rm_ss[i]
        p_s[_PH:_H, :] = jnp.full((_H - _PH, _W), -1.0, dtype=jnp.float32)
        p_s[0:_PH, _PW:_W] = jnp.full(
            (_PH, _W - _PW), -1.0, dtype=jnp.float32)
        rm_s[_PH:_H, 0:1] = jnp.full((_H - _PH, 1), -1.0, dtype=jnp.float32)
        p0 = _pool_sums(hm_refs[i][0, 0], _PH)
        p_s[0:_PH, 0:_PW] = p0
        rm_s[0:_PH, 0:1] = jnp.max(p0, axis=1, keepdims=True)

    for t in range(_N_TARGETS):
        for i in range(_PAIR):
            hm_ref, p_s, rm_s = hm_refs[i], p_ss[i], rm_ss[i]
            rm = rm_s[:, :]
            m = jnp.max(rm)
            r0 = jnp.min(jnp.where(rm == m, rowi, big))
            ra8 = (r0 // 8) * 8
            slab = p_s[pl.ds(ra8, 8), :]
            c0 = jnp.min(
                jnp.where((srow + ra8 == r0) & (slab == m), scol, big))
            out_ref[i, t, 0] = c0 + _RADIUS
            out_ref[i, t, 1] = r0 + _RADIUS

            band = hm_ref[0, 0, pl.ds(ra8, 16), :]
            br = jax.lax.broadcasted_iota(jnp.int32, (16, _W), 0) + ra8
            bc = jax.lax.broadcasted_iota(jnp.int32, (16, _W), 1)
            inside = (
                (br >= r0) & (br < r0 + _RECLEN)
                & (bc >= c0) & (bc < c0 + _RECLEN)
            )
            hm_ref[0, 0, pl.ds(ra8, 16), :] = jnp.where(inside, 0.0, band)

            pr = jnp.minimum((jnp.maximum(r0 - 8, 0) // 8) * 8, _PH - 24)
            hband = hm_ref[0, 0, pl.ds(pr, 32), :]
            pband = _pool_sums(hband, 24)
            p_s[pl.ds(pr, 24), 0:_PW] = pband
            rm_s[pl.ds(pr, 24), 0:1] = jnp.max(pband, axis=1, keepdims=True)


@jax.jit
def kernel(heatmap):
    B = heatmap.shape[0]
    out = pl.pallas_call(
        _peaks_body,
        grid=(B // _PAIR,),
        in_specs=[
            pl.BlockSpec(
                (1, 1, _H, _W),
                lambda b, i=i: (_PAIR * b + i, 0, 0, 0))
            for i in range(_PAIR)
        ],
        out_specs=pl.BlockSpec(
            (_PAIR, _N_TARGETS, 2), lambda b: (b, 0, 0),
            memory_space=pltpu.SMEM,
        ),
        out_shape=jax.ShapeDtypeStruct((B, _N_TARGETS, 2), jnp.int32),
        scratch_shapes=(
            [pltpu.VMEM((_H, _W), jnp.float32) for _ in range(_PAIR)]
            + [pltpu.VMEM((_H, 1), jnp.float32) for _ in range(_PAIR)]
        ),
        compiler_params=pltpu.CompilerParams(
            dimension_semantics=("parallel",),
        ),
    )(*([heatmap] * _PAIR))
    return out
